# Initial kernel scaffold; baseline (speedup 1.0000x reference)
import jax
import jax.numpy as jnp
from jax.experimental import pallas as pl


def kernel(inputs, indexes, features, label):
    raise NotImplementedError("write your pallas kernel here")


# speedup vs baseline: 7.9502x; 7.9502x over previous
import functools

import jax
import jax.numpy as jnp
from jax import lax
from jax.experimental import pallas as pl
from jax.experimental.pallas import tpu as pltpu
from jax.experimental.pallas import tpu_sc as plsc

NUM_SAMPLES = 100000
NUM_FEATURES = 64
NUM_CLUSTERS = 2000
C_PAD = 2048
BATCH = 1024
TEMP = 0.05

NC = 2
NS = 16
NW = NC * NS
CHUNK = 1000
NCHUNK = NUM_SAMPLES // CHUNK
SUB = 125
NSUB = CHUNK // SUB
QPW = BATCH // NW


def _sc_body(feat_hbm, lab3_hbm, labflat_hbm, idx2_hbm, zb_hbm, zc_hbm,
             ones_hbm, bank_out, cnt_out, tgt_out,
             idx_v, rows_v, ones_v, qidx_v, tgt_v, bank_sh, cnt_sh, sem):
    cid = lax.axis_index("c")
    sid = lax.axis_index("s")
    wid = sid * NC + cid

    @pl.when(sid == 0)
    def _():
        pltpu.sync_copy(zb_hbm, bank_sh)
        pltpu.sync_copy(zc_hbm, cnt_sh)

    pltpu.sync_copy(ones_hbm, ones_v)
    pltpu.sync_copy(idx2_hbm.at[wid], qidx_v)
    pltpu.async_copy(labflat_hbm.at[qidx_v], tgt_v, sem).wait()
    pltpu.sync_copy(tgt_v, tgt_out.at[wid])

    plsc.subcore_barrier()

    for t in range((NCHUNK + NW - 1) // NW):
        j = wid + NW * t

        @pl.when(j < NCHUNK)
        def _():
            pltpu.sync_copy(lab3_hbm.at[j], idx_v)
            pltpu.sync_copy(feat_hbm.at[j], rows_v)
            for r in range(NSUB):
                pltpu.sync_copy(rows_v.at[pl.ds(r * SUB, SUB), :],
                                bank_sh.at[idx_v.at[r]], add=True)
                pltpu.sync_copy(ones_v, cnt_sh.at[idx_v.at[r]], add=True)

    plsc.subcore_barrier()

    @pl.when(sid == 0)
    def _():
        pltpu.sync_copy(bank_sh, bank_out.at[cid])
        pltpu.sync_copy(cnt_sh, cnt_out.at[cid])


_sc_call = functools.partial(
    pl.kernel,
    out_type=(
        jax.ShapeDtypeStruct((NC, C_PAD, NUM_FEATURES), jnp.float32),
        jax.ShapeDtypeStruct((NC, C_PAD, 16), jnp.float32),
        jax.ShapeDtypeStruct((NW, QPW), jnp.int32),
    ),
    mesh=plsc.VectorSubcoreMesh(core_axis_name="c", subcore_axis_name="s"),
    compiler_params=pltpu.CompilerParams(use_tc_tiling_on_sc=False),
    scratch_types=(
        pltpu.VMEM((NSUB, SUB), jnp.int32),
        pltpu.VMEM((CHUNK, NUM_FEATURES), jnp.float32),
        pltpu.VMEM((SUB, 16), jnp.float32),
        pltpu.VMEM((QPW,), jnp.int32),
        pltpu.VMEM((QPW,), jnp.int32),
        pltpu.VMEM_SHARED((C_PAD, NUM_FEATURES), jnp.float32),
        pltpu.VMEM_SHARED((C_PAD, 16), jnp.float32),
        pltpu.SemaphoreType.DMA,
    ),
)(_sc_body)


def _tc_body(x_ref, bank_ref, cnt_ref, tgt_ref, out_ref):
    x = x_ref[...]
    bank = bank_ref[0] + bank_ref[1]
    cnt = cnt_ref[0, :, 0:1] + cnt_ref[1, :, 0:1]
    dots = lax.dot_general(x, bank, (((1,), (1,)), ((), ())),
                           preferred_element_type=jnp.float32,
                           precision=lax.Precision.HIGHEST)
    denom = jnp.where(cnt > 0.0, cnt, 1.0)
    scale = (1.0 / TEMP) / denom
    vec = dots * scale.T
    mask = (cnt > 0.0).astype(jnp.float32).T
    exps = jnp.exp(vec) * mask
    sums = jnp.sum(exps, axis=1, keepdims=True) + 1e-6
    cids = lax.broadcasted_iota(jnp.int32, exps.shape, 1)
    texp = jnp.sum(jnp.where(cids == tgt_ref[...], exps, 0.0),
                   axis=1, keepdims=True)
    logp = jnp.log(texp / sums + 1e-6)
    out_ref[...] = -jnp.sum(logp, axis=0, keepdims=True) / float(BATCH)


_tc_call = pl.pallas_call(
    _tc_body,
    out_shape=jax.ShapeDtypeStruct((1, 1), jnp.float32),
)


def kernel(inputs, indexes, features, label):
    feat3 = features.reshape(NCHUNK, CHUNK, NUM_FEATURES)
    lab3 = label.reshape(NCHUNK, NSUB, SUB)
    idx2 = indexes.reshape(NW, QPW)
    zb = jnp.zeros((C_PAD, NUM_FEATURES), jnp.float32)
    zc = jnp.zeros((C_PAD, 16), jnp.float32)
    ones = jnp.ones((SUB, 16), jnp.float32)
    bank2, cnt2, tgt2 = _sc_call(feat3, lab3, label, idx2, zb, zc, ones)
    tgt = tgt2.reshape(BATCH, 1)
    loss = _tc_call(inputs, bank2, cnt2, tgt)
    return loss.reshape(())


# speedup vs baseline: 8.8934x; 1.1186x over previous
import functools

import jax
import jax.numpy as jnp
from jax import lax
from jax.experimental import pallas as pl
from jax.experimental.pallas import tpu as pltpu
from jax.experimental.pallas import tpu_sc as plsc

NUM_SAMPLES = 100000
NUM_FEATURES = 64
NUM_CLUSTERS = 2000
C_PAD = 2048
BATCH = 1024
TEMP = 0.05

NC = 2
NS = 16
NW = NC * NS
SPAN = NUM_SAMPLES // NW
NCH = 5
CHUNK = SPAN // NCH
SUB = 125
NSUB = CHUNK // SUB
QPW = BATCH // NW


def _sc_body(feat_hbm, lab4_hbm, labflat_hbm, idx2_hbm, zb_hbm, zc_hbm,
             ones_hbm, bank_out, cnt_out, tgt_out,
             idx_v, rows_v, ones_v, qidx_v, tgt_v, bank_sh, cnt_sh,
             lsem, ssem, gsem):
    cid = lax.axis_index("c")
    sid = lax.axis_index("s")
    wid = sid * NC + cid
    base = wid * SPAN

    def load(c, b):
        r = pltpu.async_copy(feat_hbm.at[pl.ds(base + c * CHUNK, CHUNK), :],
                             rows_v.at[b], lsem[b])
        i = pltpu.async_copy(lab4_hbm.at[wid, c], idx_v.at[b], lsem[b])
        return (r, i)

    ld = [load(0, 0), None]

    @pl.when(sid == 0)
    def _():
        pltpu.sync_copy(zb_hbm, bank_sh)
        pltpu.sync_copy(zc_hbm, cnt_sh)

    pltpu.sync_copy(ones_hbm, ones_v)
    pltpu.sync_copy(idx2_hbm.at[wid], qidx_v)
    pltpu.async_copy(labflat_hbm.at[qidx_v], tgt_v, gsem).wait()
    pltpu.sync_copy(tgt_v, tgt_out.at[wid])

    plsc.subcore_barrier()

    scat = [[], []]
    for c in range(NCH):
        b = c & 1
        for d in ld[b]:
            d.wait()
        for r in range(NSUB):
            scat[b].append(
                pltpu.async_copy(rows_v.at[b, pl.ds(r * SUB, SUB), :],
                                 bank_sh.at[idx_v.at[b, r]], ssem[b],
                                 add=True))
            scat[b].append(
                pltpu.async_copy(ones_v, cnt_sh.at[idx_v.at[b, r]], ssem[b],
                                 add=True))
        if c + 1 < NCH:
            nb = 1 - b
            for d in scat[nb]:
                d.wait()
            scat[nb] = []
            ld[nb] = load(c + 1, nb)
    for b in (0, 1):
        for d in scat[b]:
            d.wait()

    plsc.subcore_barrier()

    @pl.when(sid == 0)
    def _():
        pltpu.sync_copy(bank_sh, bank_out.at[cid])
        pltpu.sync_copy(cnt_sh, cnt_out.at[cid])


_sc_call = functools.partial(
    pl.kernel,
    out_type=(
        jax.ShapeDtypeStruct((NC, C_PAD, NUM_FEATURES), jnp.float32),
        jax.ShapeDtypeStruct((NC, C_PAD, 16), jnp.float32),
        jax.ShapeDtypeStruct((NW, QPW), jnp.int32),
    ),
    mesh=plsc.VectorSubcoreMesh(core_axis_name="c", subcore_axis_name="s"),
    compiler_params=pltpu.CompilerParams(use_tc_tiling_on_sc=False),
    scratch_types=(
        pltpu.VMEM((2, NSUB, SUB), jnp.int32),
        pltpu.VMEM((2, CHUNK, NUM_FEATURES), jnp.float32),
        pltpu.VMEM((SUB, 16), jnp.float32),
        pltpu.VMEM((QPW,), jnp.int32),
        pltpu.VMEM((QPW,), jnp.int32),
        pltpu.VMEM_SHARED((C_PAD, NUM_FEATURES), jnp.float32),
        pltpu.VMEM_SHARED((C_PAD, 16), jnp.float32),
        (pltpu.SemaphoreType.DMA, pltpu.SemaphoreType.DMA),
        (pltpu.SemaphoreType.DMA, pltpu.SemaphoreType.DMA),
        pltpu.SemaphoreType.DMA,
    ),
)(_sc_body)


def _tc_body(x_ref, bank_ref, cnt_ref, tgt_ref, out_ref):
    x = x_ref[...]
    bank = bank_ref[0] + bank_ref[1]
    cnt = cnt_ref[0, :, 0:1] + cnt_ref[1, :, 0:1]
    dots = lax.dot_general(x, bank, (((1,), (1,)), ((), ())),
                           preferred_element_type=jnp.float32,
                           precision=lax.Precision.HIGHEST)
    denom = jnp.where(cnt > 0.0, cnt, 1.0)
    scale = (1.0 / TEMP) / denom
    vec = dots * scale.T
    mask = (cnt > 0.0).astype(jnp.float32).T
    exps = jnp.exp(vec) * mask
    sums = jnp.sum(exps, axis=1, keepdims=True) + 1e-6
    cids = lax.broadcasted_iota(jnp.int32, exps.shape, 1)
    texp = jnp.sum(jnp.where(cids == tgt_ref[...], exps, 0.0),
                   axis=1, keepdims=True)
    logp = jnp.log(texp / sums + 1e-6)
    out_ref[...] = -jnp.sum(logp, axis=0, keepdims=True) / float(BATCH)


_tc_call = pl.pallas_call(
    _tc_body,
    out_shape=jax.ShapeDtypeStruct((1, 1), jnp.float32),
)


def kernel(inputs, indexes, features, label):
    lab4 = label.reshape(NW, NCH, NSUB, SUB)
    idx2 = indexes.reshape(NW, QPW)
    zb = jnp.zeros((C_PAD, NUM_FEATURES), jnp.float32)
    zc = jnp.zeros((C_PAD, 16), jnp.float32)
    ones = jnp.ones((SUB, 16), jnp.float32)
    bank2, cnt2, tgt2 = _sc_call(features, lab4, label, idx2, zb, zc, ones)
    tgt = tgt2.reshape(BATCH, 1)
    loss = _tc_call(inputs, bank2, cnt2, tgt)
    return loss.reshape(())
